# Initial kernel scaffold; baseline (speedup 1.0000x reference)
import jax
import jax.numpy as jnp
from jax.experimental import pallas as pl


def kernel(in_features, W1, W2, weight, node_ids2, neigh_pos2, cur_pos2, neigh_pos1, cur_pos1):
    raise NotImplementedError("write your pallas kernel here")


# speedup vs baseline: 1.5980x; 1.5980x over previous
import functools

import jax
import jax.numpy as jnp
from jax import lax
from jax.experimental import pallas as pl
from jax.experimental.pallas import tpu as pltpu
from jax.experimental.pallas import tpu_sc as plsc

_NC = 2
_NS = 16
_NW = _NC * _NS
_LANES = 16
_FANOUT = 32
_GRP = 4
_GRP_ROWS = _GRP * _FANOUT


def _sc_gather_mean(table, nidx, cidx, nid=None):
    t_rows, d = table.shape
    nw, n_g, _ = nidx.shape
    n_c = cidx.shape[1]
    r = n_g * _GRP
    assert d == 128 and r == n_c * 64 and n_g % 2 == 0

    compose = nid is not None
    mesh = plsc.VectorSubcoreMesh(core_axis_name="c", subcore_axis_name="s")

    scratch = [
        pltpu.VMEM((n_g, _GRP_ROWS), jnp.int32),
        pltpu.VMEM((n_c, 64), jnp.int32),
        pltpu.VMEM((_GRP_ROWS, d), jnp.float32),
        pltpu.VMEM((_GRP_ROWS, d), jnp.float32),
        pltpu.VMEM((r, d), jnp.float32),
        pltpu.SemaphoreType.DMA,
        pltpu.SemaphoreType.DMA,
        pltpu.SemaphoreType.DMA,
    ]
    if compose:
        scratch.append(pltpu.VMEM((nid.shape[0],), jnp.int32))

    @functools.partial(
        pl.kernel,
        out_type=[
            jax.ShapeDtypeStruct((nw * r, d), jnp.float32),
            jax.ShapeDtypeStruct((nw * r, d), jnp.float32),
        ],
        mesh=mesh,
        scratch_types=scratch,
        compiler_params=pltpu.CompilerParams(needs_layout_passes=False),
    )
    def run(*args):
        if compose:
            (table_h, nidx_h, cidx_h, nid_h, x_h, agg_h,
             nidx_v, cidx_v, buf0, buf1, rows_v, sem0, sem1, xsem,
             nid_v) = args
        else:
            (table_h, nidx_h, cidx_h, x_h, agg_h,
             nidx_v, cidx_v, buf0, buf1, rows_v, sem0, sem1, xsem) = args
        bufs = (buf0, buf1)
        sems = (sem0, sem1)
        wid = lax.axis_index("s") * _NC + lax.axis_index("c")
        base = wid * r

        pltpu.sync_copy(nidx_h.at[wid], nidx_v)
        pltpu.sync_copy(cidx_h.at[wid], cidx_v)
        if compose:
            pltpu.sync_copy(nid_h, nid_v)

            def compose_row(g, _):
                for v in range(_GRP_ROWS // _LANES):
                    sl = pl.ds(v * _LANES, _LANES)
                    nidx_v[g, sl] = plsc.load_gather(nid_v, [nidx_v[g, sl]])
                return _

            lax.fori_loop(0, n_g, compose_row, None, unroll=False)
            for c in range(n_c):
                for v in range(64 // _LANES):
                    sl = pl.ds(v * _LANES, _LANES)
                    cidx_v[c, sl] = plsc.load_gather(nid_v, [cidx_v[c, sl]])

        pltpu.async_copy(table_h.at[nidx_v.at[0]], buf0, sem0)
        pltpu.async_copy(table_h.at[nidx_v.at[1]], buf1, sem1)

        xh = [
            pltpu.async_copy(
                table_h.at[cidx_v.at[c]],
                rows_v.at[pl.ds(c * 64, 64)], xsem)
            for c in range(n_c)
        ]
        for h in xh:
            h.wait()
        pltpu.sync_copy(rows_v, x_h.at[pl.ds(base, r)])

        def reduce_group(g, buf):
            for dd in range(_GRP):
                row0 = dd * _FANOUT
                acc = [buf[row0, pl.ds(j * _LANES, _LANES)] for j in range(8)]
                for i in range(1, _FANOUT):
                    for j in range(8):
                        acc[j] = acc[j] + buf[row0 + i, pl.ds(j * _LANES, _LANES)]
                for j in range(8):
                    rows_v[g * _GRP + dd, pl.ds(j * _LANES, _LANES)] = (
                        acc[j] * (1.0 / _FANOUT))

        def ring_wait(buf, sem):
            pltpu.make_async_copy(table_h.at[pl.ds(0, _GRP_ROWS)], buf, sem).wait()

        def main_body(i, _):
            g0 = i * 2
            for b in range(2):
                g = g0 + b
                ring_wait(bufs[b], sems[b])
                reduce_group(g, bufs[b])
                pltpu.async_copy(table_h.at[nidx_v.at[g + 2]], bufs[b], sems[b])
            return _

        lax.fori_loop(0, (n_g - 2) // 2, main_body, None, unroll=False)
        for b in range(2):
            g = n_g - 2 + b
            ring_wait(bufs[b], sems[b])
            reduce_group(g, bufs[b])

        pltpu.sync_copy(rows_v, agg_h.at[pl.ds(base, r)])

    if compose:
        return run(table, nidx, cidx, nid)
    return run(table, nidx, cidx)


def _tc_sage_layer(x, agg, w_top, w_bot, blk):
    m, d = x.shape
    h = w_top.shape[1]

    def body(x_r, a_r, wt_r, wb_r, o_r):
        o_r[...] = jnp.maximum(
            jnp.dot(x_r[...], wt_r[...], preferred_element_type=jnp.float32)
            + jnp.dot(a_r[...], wb_r[...], preferred_element_type=jnp.float32),
            0.0)

    return pl.pallas_call(
        body,
        grid=(m // blk,),
        in_specs=[
            pl.BlockSpec((blk, d), lambda i: (i, 0)),
            pl.BlockSpec((blk, d), lambda i: (i, 0)),
            pl.BlockSpec((d, h), lambda i: (0, 0)),
            pl.BlockSpec((d, h), lambda i: (0, 0)),
        ],
        out_specs=pl.BlockSpec((blk, h), lambda i: (i, 0)),
        out_shape=jax.ShapeDtypeStruct((m, h), jnp.float32),
    )(x, agg, w_top, w_bot)


def _tc_final(x, agg, w_top, w_bot, proj):
    m = x.shape[0]
    h = w_top.shape[1]
    out = proj.shape[1]

    def body(x_r, a_r, wt_r, wb_r, p_r, o_r):
        hid = jnp.maximum(
            jnp.dot(x_r[...], wt_r[...], preferred_element_type=jnp.float32)
            + jnp.dot(a_r[...], wb_r[...], preferred_element_type=jnp.float32),
            0.0)
        o_r[...] = jax.nn.sigmoid(
            jnp.dot(hid, p_r[...], preferred_element_type=jnp.float32))

    return pl.pallas_call(
        body,
        out_shape=jax.ShapeDtypeStruct((m, out), jnp.float32),
    )(x, agg, w_top, w_bot, proj)


def _pad_indices(nidx, cidx, r):
    u = nidx.shape[0]
    up = _NW * r
    nidx = jnp.pad(nidx, ((0, up - u), (0, 0)))
    cidx = jnp.pad(cidx, (0, up - u))
    nidx = nidx.reshape(_NW, (r * _FANOUT) // _GRP_ROWS, _GRP_ROWS)
    cidx = cidx.reshape(_NW, r // 64, 64)
    return nidx, cidx


def kernel(in_features, W1, W2, weight, node_ids2, neigh_pos2, cur_pos2,
           neigh_pos1, cur_pos1):
    d = in_features.shape[1]
    u1 = neigh_pos2.shape[0]
    b = neigh_pos1.shape[0]

    r1 = -((-u1) // (_NW * 64)) * 64
    r2 = -((-b) // (_NW * 64)) * 64

    nidx2, cidx2 = _pad_indices(neigh_pos2, cur_pos2, r1)
    nidx1, cidx1 = _pad_indices(neigh_pos1, cur_pos1, r2)

    x2, agg2 = _sc_gather_mean(in_features, nidx2, cidx2, nid=node_ids2)
    h1 = _tc_sage_layer(x2, agg2, W1[:d], W1[d:], blk=1024)

    x1, agg1 = _sc_gather_mean(h1, nidx1, cidx1)
    hd = W2.shape[1]
    out = _tc_final(x1[:b], agg1[:b], W2[:hd], W2[hd:], weight)
    return out


# speedup vs baseline: 1.6784x; 1.0504x over previous
import functools

import jax
import jax.numpy as jnp
from jax import lax
from jax.experimental import pallas as pl
from jax.experimental.pallas import tpu as pltpu
from jax.experimental.pallas import tpu_sc as plsc

_NC = 2
_NS = 16
_NW = _NC * _NS
_LANES = 16
_FANOUT = 32
_GRP = 4
_GRP_ROWS = _GRP * _FANOUT
_NBUF = 4


def _sc_gather_mean(table, nidx, cidx, nid=None):
    t_rows, d = table.shape
    nw, n_g, _ = nidx.shape
    n_c = cidx.shape[1]
    r = n_g * _GRP
    n_it = n_g // _NBUF
    s_rows = _NBUF * _GRP
    assert d == 128 and r == n_c * 64
    assert n_g % _NBUF == 0 and n_it % 2 == 0 and n_it >= 4

    compose = nid is not None
    mesh = plsc.VectorSubcoreMesh(core_axis_name="c", subcore_axis_name="s")

    scratch = [
        pltpu.VMEM((n_g, _GRP_ROWS), jnp.int32),
        pltpu.VMEM((n_c, 64), jnp.int32),
    ]
    scratch += [pltpu.VMEM((_GRP_ROWS, d), jnp.float32)] * _NBUF
    scratch += [pltpu.VMEM((s_rows, d), jnp.float32)] * 2
    scratch += [pltpu.VMEM((64, d), jnp.float32)] * 2
    scratch += [pltpu.SemaphoreType.DMA] * (_NBUF + 4)
    if compose:
        scratch.append(pltpu.VMEM((nid.shape[0],), jnp.int32))

    @functools.partial(
        pl.kernel,
        out_type=[
            jax.ShapeDtypeStruct((nw * r, d), jnp.float32),
            jax.ShapeDtypeStruct((nw * r, d), jnp.float32),
        ],
        mesh=mesh,
        scratch_types=scratch,
        compiler_params=pltpu.CompilerParams(needs_layout_passes=False),
    )
    def run(*args):
        n_in = 4 if compose else 3
        table_h, nidx_h, cidx_h = args[:3]
        x_h, agg_h = args[n_in:n_in + 2]
        a = n_in + 2
        nidx_v, cidx_v = args[a:a + 2]
        bufs = args[a + 2:a + 2 + _NBUF]
        stage = args[a + 2 + _NBUF:a + 4 + _NBUF]
        xbuf = args[a + 4 + _NBUF:a + 6 + _NBUF]
        rsems = args[a + 6 + _NBUF:a + 6 + 2 * _NBUF]
        asems = args[a + 6 + 2 * _NBUF:a + 8 + 2 * _NBUF]
        xsems = args[a + 8 + 2 * _NBUF:a + 10 + 2 * _NBUF]
        if compose:
            nid_h = args[3]
            nid_v = args[-1]

        wid = lax.axis_index("s") * _NC + lax.axis_index("c")
        base = wid * r

        pltpu.sync_copy(nidx_h.at[wid], nidx_v)
        pltpu.sync_copy(cidx_h.at[wid], cidx_v)

        def compose_row(g):
            for v in range(_GRP_ROWS // _LANES):
                sl = pl.ds(v * _LANES, _LANES)
                nidx_v[g, sl] = plsc.load_gather(nid_v, [nidx_v[g, sl]])

        if compose:
            pltpu.sync_copy(nid_h, nid_v)
            for g in range(_NBUF):
                compose_row(g)

        def issue(g, b):
            pltpu.async_copy(table_h.at[nidx_v.at[g]], bufs[b], rsems[b])

        for b in range(_NBUF):
            issue(b, b)

        if compose:
            for c in range(n_c):
                for v in range(64 // _LANES):
                    sl = pl.ds(v * _LANES, _LANES)
                    cidx_v[c, sl] = plsc.load_gather(nid_v, [cidx_v[c, sl]])

        def xissue(c):
            pltpu.async_copy(table_h.at[cidx_v.at[c]], xbuf[c % 2],
                             xsems[c % 2])

        for c in range(min(2, n_c)):
            xissue(c)

        if compose:
            def compose_rest(g, _):
                compose_row(g)
                return _
            lax.fori_loop(_NBUF, n_g, compose_rest, None, unroll=False)

        for c in range(n_c):
            pltpu.make_async_copy(
                table_h.at[cidx_v.at[c]], xbuf[c % 2], xsems[c % 2]).wait()
            pltpu.sync_copy(xbuf[c % 2], x_h.at[pl.ds(base + c * 64, 64)])
            if c + 2 < n_c:
                xissue(c + 2)

        def reduce_group(buf, st, b):
            def dest_body(dd, _):
                row0 = dd * _FANOUT
                acc = [jnp.full((_LANES,), 0.0, jnp.float32)
                       for _j in range(8)]

                def octet(k, acc):
                    r0 = row0 + k * 8
                    for i in range(8):
                        acc = [acc[j] + buf[r0 + i, pl.ds(j * _LANES, _LANES)]
                               for j in range(8)]
                    return acc

                acc = lax.fori_loop(0, _FANOUT // 8, octet, acc, unroll=False)
                for j in range(8):
                    st[b * _GRP + dd, pl.ds(j * _LANES, _LANES)] = (
                        acc[j] * (1.0 / _FANOUT))
                return _

            lax.fori_loop(0, _GRP, dest_body, None, unroll=False)

        def ring_wait(b):
            pltpu.make_async_copy(
                table_h.at[pl.ds(0, _GRP_ROWS)], bufs[b], rsems[b]).wait()

        def agg_wait(p):
            pltpu.make_async_copy(
                stage[p], agg_h.at[pl.ds(0, s_rows)], asems[p]).wait()

        def do_iter(i, p, do_issue, wait_agg):
            if wait_agg:
                agg_wait(p)
            for b in range(_NBUF):
                g = i * _NBUF + b
                ring_wait(b)
                reduce_group(bufs[b], stage[p], b)
                if do_issue:
                    issue(g + _NBUF, b)
            pltpu.async_copy(stage[p],
                             agg_h.at[pl.ds(base + i * s_rows, s_rows)],
                             asems[p])

        do_iter(0, 0, True, False)
        do_iter(1, 1, True, False)

        def pair_body(j, _):
            do_iter(2 * j, 0, True, True)
            do_iter(2 * j + 1, 1, True, True)
            return _

        lax.fori_loop(1, n_it // 2 - 1, pair_body, None, unroll=False)
        do_iter(n_it - 2, 0, True, True)
        do_iter(n_it - 1, 1, False, True)
        agg_wait(0)
        agg_wait(1)

    if compose:
        return run(table, nidx, cidx, nid)
    return run(table, nidx, cidx)


def _tc_sage_layer(x, agg, w_top, w_bot, blk):
    m, d = x.shape
    h = w_top.shape[1]

    def body(x_r, a_r, wt_r, wb_r, o_r):
        o_r[...] = jnp.maximum(
            jnp.dot(x_r[...], wt_r[...], preferred_element_type=jnp.float32)
            + jnp.dot(a_r[...], wb_r[...], preferred_element_type=jnp.float32),
            0.0)

    return pl.pallas_call(
        body,
        grid=(m // blk,),
        in_specs=[
            pl.BlockSpec((blk, d), lambda i: (i, 0)),
            pl.BlockSpec((blk, d), lambda i: (i, 0)),
            pl.BlockSpec((d, h), lambda i: (0, 0)),
            pl.BlockSpec((d, h), lambda i: (0, 0)),
        ],
        out_specs=pl.BlockSpec((blk, h), lambda i: (i, 0)),
        out_shape=jax.ShapeDtypeStruct((m, h), jnp.float32),
    )(x, agg, w_top, w_bot)


def _tc_final(x, agg, w_top, w_bot, proj):
    m = x.shape[0]
    h = w_top.shape[1]
    out = proj.shape[1]

    def body(x_r, a_r, wt_r, wb_r, p_r, o_r):
        hid = jnp.maximum(
            jnp.dot(x_r[...], wt_r[...], preferred_element_type=jnp.float32)
            + jnp.dot(a_r[...], wb_r[...], preferred_element_type=jnp.float32),
            0.0)
        o_r[...] = jax.nn.sigmoid(
            jnp.dot(hid, p_r[...], preferred_element_type=jnp.float32))

    return pl.pallas_call(
        body,
        out_shape=jax.ShapeDtypeStruct((m, out), jnp.float32),
    )(x, agg, w_top, w_bot, proj)


def _pad_indices(nidx, cidx, r):
    u = nidx.shape[0]
    up = _NW * r
    nidx = jnp.pad(nidx, ((0, up - u), (0, 0)))
    cidx = jnp.pad(cidx, (0, up - u))
    nidx = nidx.reshape(_NW, (r * _FANOUT) // _GRP_ROWS, _GRP_ROWS)
    cidx = cidx.reshape(_NW, r // 64, 64)
    return nidx, cidx


def kernel(in_features, W1, W2, weight, node_ids2, neigh_pos2, cur_pos2,
           neigh_pos1, cur_pos1):
    d = in_features.shape[1]
    u1 = neigh_pos2.shape[0]
    b = neigh_pos1.shape[0]

    r1 = -((-u1) // (_NW * 64)) * 64
    r2 = -((-b) // (_NW * 64)) * 64

    nidx2, cidx2 = _pad_indices(neigh_pos2, cur_pos2, r1)
    nidx1, cidx1 = _pad_indices(neigh_pos1, cur_pos1, r2)

    x2, agg2 = _sc_gather_mean(in_features, nidx2, cidx2, nid=node_ids2)
    h1 = _tc_sage_layer(x2, agg2, W1[:d], W1[d:], blk=1024)

    x1, agg1 = _sc_gather_mean(h1, nidx1, cidx1)
    hd = W2.shape[1]
    out = _tc_final(x1[:b], agg1[:b], W2[:hd], W2[hd:], weight)
    return out


# speedup vs baseline: 1.6788x; 1.0002x over previous
import functools

import jax
import jax.numpy as jnp
from jax import lax
from jax.experimental import pallas as pl
from jax.experimental.pallas import tpu as pltpu
from jax.experimental.pallas import tpu_sc as plsc

_NC = 2
_NS = 16
_NW = _NC * _NS
_LANES = 16
_FANOUT = 32
_GRP = 4
_GRP_ROWS = _GRP * _FANOUT
_NBUF = 4


def _sc_gather_mean(table, nidx, cidx, nid=None):
    t_rows, d = table.shape
    nw, n_g, _ = nidx.shape
    n_c = cidx.shape[1]
    r = n_g * _GRP
    n_it = n_g // _NBUF
    s_rows = _NBUF * _GRP
    assert d == 128 and r == n_c * 64
    assert n_g % _NBUF == 0 and n_it % 2 == 0 and n_it >= 4

    compose = nid is not None
    mesh = plsc.VectorSubcoreMesh(core_axis_name="c", subcore_axis_name="s")

    scratch = [
        pltpu.VMEM((n_g, _GRP_ROWS), jnp.int32),
        pltpu.VMEM((n_c, 64), jnp.int32),
    ]
    scratch += [pltpu.VMEM((_GRP_ROWS, d), jnp.float32)] * _NBUF
    scratch += [pltpu.VMEM((s_rows, d), jnp.float32)] * 2
    scratch += [pltpu.VMEM((64, d), jnp.float32)] * 2
    scratch += [pltpu.SemaphoreType.DMA] * (_NBUF + 4)
    if compose:
        scratch.append(pltpu.VMEM((nid.shape[0],), jnp.int32))

    @functools.partial(
        pl.kernel,
        out_type=[
            jax.ShapeDtypeStruct((nw * r, d), jnp.float32),
            jax.ShapeDtypeStruct((nw * r, d), jnp.float32),
        ],
        mesh=mesh,
        scratch_types=scratch,
        compiler_params=pltpu.CompilerParams(needs_layout_passes=False),
    )
    def run(*args):
        n_in = 4 if compose else 3
        table_h, nidx_h, cidx_h = args[:3]
        x_h, agg_h = args[n_in:n_in + 2]
        a = n_in + 2
        nidx_v, cidx_v = args[a:a + 2]
        bufs = args[a + 2:a + 2 + _NBUF]
        stage = args[a + 2 + _NBUF:a + 4 + _NBUF]
        xbuf = args[a + 4 + _NBUF:a + 6 + _NBUF]
        rsems = args[a + 6 + _NBUF:a + 6 + 2 * _NBUF]
        asems = args[a + 6 + 2 * _NBUF:a + 8 + 2 * _NBUF]
        xsems = args[a + 8 + 2 * _NBUF:a + 10 + 2 * _NBUF]
        if compose:
            nid_h = args[3]
            nid_v = args[-1]

        wid = lax.axis_index("s") * _NC + lax.axis_index("c")
        base = wid * r

        scope = jax.named_scope("sc_prolog")
        scope.__enter__()
        pltpu.sync_copy(nidx_h.at[wid], nidx_v)
        pltpu.sync_copy(cidx_h.at[wid], cidx_v)

        def compose_row(g):
            for v in range(_GRP_ROWS // _LANES):
                sl = pl.ds(v * _LANES, _LANES)
                nidx_v[g, sl] = plsc.load_gather(nid_v, [nidx_v[g, sl]])

        if compose:
            pltpu.sync_copy(nid_h, nid_v)
            for g in range(_NBUF):
                compose_row(g)

        def issue(g, b):
            pltpu.async_copy(table_h.at[nidx_v.at[g]], bufs[b], rsems[b])

        for b in range(_NBUF):
            issue(b, b)

        if compose:
            for c in range(n_c):
                for v in range(64 // _LANES):
                    sl = pl.ds(v * _LANES, _LANES)
                    cidx_v[c, sl] = plsc.load_gather(nid_v, [cidx_v[c, sl]])

        def xissue(c):
            pltpu.async_copy(table_h.at[cidx_v.at[c]], xbuf[c % 2],
                             xsems[c % 2])

        for c in range(min(2, n_c)):
            xissue(c)

        if compose:
            def compose_rest(g, _):
                compose_row(g)
                return _
            lax.fori_loop(_NBUF, n_g, compose_rest, None, unroll=False)

        scope.__exit__(None, None, None)
        scope = jax.named_scope("sc_xphase")
        scope.__enter__()
        for c in range(n_c):
            pltpu.make_async_copy(
                table_h.at[cidx_v.at[c]], xbuf[c % 2], xsems[c % 2]).wait()
            pltpu.sync_copy(xbuf[c % 2], x_h.at[pl.ds(base + c * 64, 64)])
            if c + 2 < n_c:
                xissue(c + 2)
        scope.__exit__(None, None, None)
        scope = jax.named_scope("sc_main")
        scope.__enter__()

        def reduce_group(buf, st, b):
            def dest_body(dd, _):
                row0 = dd * _FANOUT
                acc = [jnp.full((_LANES,), 0.0, jnp.float32)
                       for _j in range(8)]

                def octet(k, acc):
                    r0 = row0 + k * 8
                    for i in range(8):
                        acc = [acc[j] + buf[r0 + i, pl.ds(j * _LANES, _LANES)]
                               for j in range(8)]
                    return acc

                acc = lax.fori_loop(0, _FANOUT // 8, octet, acc, unroll=False)
                for j in range(8):
                    st[b * _GRP + dd, pl.ds(j * _LANES, _LANES)] = (
                        acc[j] * (1.0 / _FANOUT))
                return _

            lax.fori_loop(0, _GRP, dest_body, None, unroll=False)

        def ring_wait(b):
            pltpu.make_async_copy(
                table_h.at[pl.ds(0, _GRP_ROWS)], bufs[b], rsems[b]).wait()

        def agg_wait(p):
            pltpu.make_async_copy(
                stage[p], agg_h.at[pl.ds(0, s_rows)], asems[p]).wait()

        def do_iter(i, p, do_issue, wait_agg):
            if wait_agg:
                agg_wait(p)
            for b in range(_NBUF):
                g = i * _NBUF + b
                ring_wait(b)
                reduce_group(bufs[b], stage[p], b)
                if do_issue:
                    issue(g + _NBUF, b)
            pltpu.async_copy(stage[p],
                             agg_h.at[pl.ds(base + i * s_rows, s_rows)],
                             asems[p])

        do_iter(0, 0, True, False)
        do_iter(1, 1, True, False)

        def pair_body(j, _):
            do_iter(2 * j, 0, True, True)
            do_iter(2 * j + 1, 1, True, True)
            return _

        lax.fori_loop(1, n_it // 2 - 1, pair_body, None, unroll=False)
        do_iter(n_it - 2, 0, True, True)
        do_iter(n_it - 1, 1, False, True)
        agg_wait(0)
        agg_wait(1)
        scope.__exit__(None, None, None)

    if compose:
        return run(table, nidx, cidx, nid)
    return run(table, nidx, cidx)


def _tc_sage_layer(x, agg, w_top, w_bot, blk):
    m, d = x.shape
    h = w_top.shape[1]

    def body(x_r, a_r, wt_r, wb_r, o_r):
        o_r[...] = jnp.maximum(
            jnp.dot(x_r[...], wt_r[...], preferred_element_type=jnp.float32)
            + jnp.dot(a_r[...], wb_r[...], preferred_element_type=jnp.float32),
            0.0)

    return pl.pallas_call(
        body,
        grid=(m // blk,),
        in_specs=[
            pl.BlockSpec((blk, d), lambda i: (i, 0)),
            pl.BlockSpec((blk, d), lambda i: (i, 0)),
            pl.BlockSpec((d, h), lambda i: (0, 0)),
            pl.BlockSpec((d, h), lambda i: (0, 0)),
        ],
        out_specs=pl.BlockSpec((blk, h), lambda i: (i, 0)),
        out_shape=jax.ShapeDtypeStruct((m, h), jnp.float32),
    )(x, agg, w_top, w_bot)


def _tc_final(x, agg, w_top, w_bot, proj):
    m = x.shape[0]
    h = w_top.shape[1]
    out = proj.shape[1]

    def body(x_r, a_r, wt_r, wb_r, p_r, o_r):
        hid = jnp.maximum(
            jnp.dot(x_r[...], wt_r[...], preferred_element_type=jnp.float32)
            + jnp.dot(a_r[...], wb_r[...], preferred_element_type=jnp.float32),
            0.0)
        o_r[...] = jax.nn.sigmoid(
            jnp.dot(hid, p_r[...], preferred_element_type=jnp.float32))

    return pl.pallas_call(
        body,
        out_shape=jax.ShapeDtypeStruct((m, out), jnp.float32),
    )(x, agg, w_top, w_bot, proj)


def _pad_indices(nidx, cidx, r):
    u = nidx.shape[0]
    up = _NW * r
    nidx = jnp.pad(nidx, ((0, up - u), (0, 0)))
    cidx = jnp.pad(cidx, (0, up - u))
    nidx = nidx.reshape(_NW, (r * _FANOUT) // _GRP_ROWS, _GRP_ROWS)
    cidx = cidx.reshape(_NW, r // 64, 64)
    return nidx, cidx


def kernel(in_features, W1, W2, weight, node_ids2, neigh_pos2, cur_pos2,
           neigh_pos1, cur_pos1):
    d = in_features.shape[1]
    u1 = neigh_pos2.shape[0]
    b = neigh_pos1.shape[0]

    r1 = -((-u1) // (_NW * 64)) * 64
    r2 = -((-b) // (_NW * 64)) * 64

    nidx2, cidx2 = _pad_indices(neigh_pos2, cur_pos2, r1)
    nidx1, cidx1 = _pad_indices(neigh_pos1, cur_pos1, r2)

    x2, agg2 = _sc_gather_mean(in_features, nidx2, cidx2, nid=node_ids2)
    h1 = _tc_sage_layer(x2, agg2, W1[:d], W1[d:], blk=1024)

    x1, agg1 = _sc_gather_mean(h1, nidx1, cidx1)
    hd = W2.shape[1]
    out = _tc_final(x1[:b], agg1[:b], W2[:hd], W2[hd:], weight)
    return out
